# baseline (speedup 1.0000x reference)
import functools

import jax
import jax.numpy as jnp
from jax.experimental import pallas as pl
from jax.experimental.pallas import tpu as pltpu

NUM_LEVEL = 3
B = 4
N = 2048
D = 256
HID = 256
TDIM = 128
K = 16
RB = 2048
NB = N // RB


def _body(x_blk_ref, x_full_ref, theta_ref, bias_ref, wfc_ref, bfc_ref,
          out_ref, sq_scr, xt_scr, z_scr, dv_scr):
    l = pl.program_id(1)
    ib = pl.program_id(2)

    x_full = x_full_ref[0, 0]
    x_blk = x_blk_ref[0, 0]

    @pl.when(ib == 0)
    def _init():
        x2 = x_full * x_full
        ones8 = jnp.ones((8, D), jnp.float32)
        sqs = jax.lax.dot_general(
            ones8, x2, (((1,), (1,)), ((), ())),
            preferred_element_type=jnp.float32)
        sq_scr[...] = sqs[0:1]
        xt_scr[...] = jnp.dot(x_full, theta_ref[0],
                              preferred_element_type=jnp.float32)
        z_scr[...] = jnp.zeros_like(z_scr)
        dv_scr[...] = jnp.zeros_like(dv_scr)

    g = jax.lax.dot_general(
        x_blk, x_full, (((1,), (1,)), ((), ())),
        preferred_element_type=jnp.float32)
    sq_blk = jnp.sum(x_blk * x_blk, axis=1, keepdims=True)
    dis = (sq_blk + sq_scr[...]) - 2.0 * g

    rows = jax.lax.broadcasted_iota(jnp.int32, dis.shape, 0) + ib * RB
    cols = jax.lax.broadcasted_iota(jnp.int32, dis.shape, 1)
    inf = jnp.float32(jnp.inf)
    work = jnp.where(rows == cols, inf, dis)
    for _ in range(K - 1):
        m = jnp.min(work, axis=1, keepdims=True)
        work = jnp.where(work <= m, inf, work)
    s = (work == inf).astype(jnp.float32)

    y = jnp.dot(s, xt_scr[...], preferred_element_type=jnp.float32)
    z_scr[...] += jax.lax.dot_general(
        s, y, (((0,), (0,)), ((), ())),
        preferred_element_type=jnp.float32)
    ones_dv = jnp.ones((RB, 128), jnp.float32)
    dv_scr[...] += jax.lax.dot_general(
        s, ones_dv, (((0,), (0,)), ((), ())),
        preferred_element_type=jnp.float32)

    @pl.when(ib == NB - 1)
    def _finish():
        dv = dv_scr[:, 0:1]
        z = z_scr[...]
        o = jnp.where(dv > 0.0, z / (jnp.float32(K) * dv), 0.0)
        o = o + bias_ref[0]
        h = jnp.where(o >= 0.0, o, jnp.float32(0.01) * o)
        feat = jnp.mean(h, axis=0, keepdims=True)
        contrib = jnp.dot(feat, wfc_ref[0],
                          preferred_element_type=jnp.float32)

        @pl.when(l == 0)
        def _first():
            out_ref[0] = bfc_ref[0] + contrib

        @pl.when(l > 0)
        def _rest():
            out_ref[0] += contrib


@jax.jit
def _run(xs, thetas, biases, wfc, bfc):
    grid = (B, NUM_LEVEL, NB)
    return pl.pallas_call(
        _body,
        grid=grid,
        in_specs=[
            pl.BlockSpec((1, 1, RB, D), lambda b, l, ib: (l, b, ib, 0)),
            pl.BlockSpec((1, 1, N, D), lambda b, l, ib: (l, b, 0, 0)),
            pl.BlockSpec((1, D, HID), lambda b, l, ib: (l, 0, 0)),
            pl.BlockSpec((1, 1, HID), lambda b, l, ib: (l, 0, 0)),
            pl.BlockSpec((1, HID, TDIM), lambda b, l, ib: (l, 0, 0)),
            pl.BlockSpec((1, 1, TDIM), lambda b, l, ib: (0, 0, 0)),
        ],
        out_specs=pl.BlockSpec((1, 1, TDIM), lambda b, l, ib: (b, 0, 0)),
        out_shape=jax.ShapeDtypeStruct((B, 1, TDIM), jnp.float32),
        scratch_shapes=[
            pltpu.VMEM((1, N), jnp.float32),
            pltpu.VMEM((N, HID), jnp.float32),
            pltpu.VMEM((N, HID), jnp.float32),
            pltpu.VMEM((N, 128), jnp.float32),
        ],
    )(xs, xs, thetas, biases, wfc, bfc)


def kernel(x0, x1, x2, c0, c1, c2, theta0, bias0, theta1, bias1,
           theta2, bias2, W_fc, b_fc):
    xs = jnp.stack([x0, x1, x2])
    thetas = jnp.stack([theta0, theta1, theta2])
    biases = jnp.stack([bias0, bias1, bias2]).reshape(NUM_LEVEL, 1, HID)
    wfc = W_fc.reshape(NUM_LEVEL, HID, TDIM)
    bfc = b_fc.reshape(1, 1, TDIM)
    out = _run(xs, thetas, biases, wfc, bfc)
    return out.reshape(B, TDIM)


# speedup vs baseline: 1.3596x; 1.3596x over previous
import functools

import jax
import jax.numpy as jnp
from jax.experimental import pallas as pl
from jax.experimental.pallas import tpu as pltpu

NUM_LEVEL = 3
B = 4
N = 2048
D = 256
HID = 256
TDIM = 128
K = 16
RB = 1024
NB = N // RB


def _body(x_blk_ref, x_full_ref, theta_ref, bias_ref, wfc_ref, bfc_ref,
          out_ref, sq_scr, xt_scr, z_scr, dv_scr):
    l = pl.program_id(1)
    ib = pl.program_id(2)

    x_full = x_full_ref[0, 0]
    x_blk = x_blk_ref[0, 0]

    @pl.when(ib == 0)
    def _init():
        x2 = x_full * x_full
        ones8 = jnp.ones((8, D), jnp.float32)
        sqs = jax.lax.dot_general(
            ones8, x2, (((1,), (1,)), ((), ())),
            preferred_element_type=jnp.float32)
        sq_scr[...] = sqs[0:1]
        xt_scr[...] = jnp.dot(x_full, theta_ref[0],
                              preferred_element_type=jnp.float32)
        z_scr[...] = jnp.zeros_like(z_scr)
        dv_scr[...] = jnp.zeros_like(dv_scr)

    g = jax.lax.dot_general(
        x_blk, x_full, (((1,), (1,)), ((), ())),
        preferred_element_type=jnp.float32)
    dis = sq_scr[...] - 2.0 * g

    rows = jax.lax.broadcasted_iota(jnp.int32, dis.shape, 0) + ib * RB
    cols = jax.lax.broadcasted_iota(jnp.int32, dis.shape, 1)
    inf = jnp.float32(jnp.inf)
    work = jnp.where(rows == cols, inf, dis)
    a = work[:, :N // 2]
    b2 = work[:, N // 2:]
    p = jnp.minimum(a, b2)
    mm = jnp.maximum(a, b2)
    m = jnp.min(p, axis=1, keepdims=True)
    for _ in range(K - 2):
        p = jnp.where(p <= m, jnp.where(p >= mm, inf, mm), p)
        m = jnp.min(p, axis=1, keepdims=True)
    s = (dis <= m).astype(jnp.float32)

    y = jnp.dot(s, xt_scr[...], preferred_element_type=jnp.float32)
    z_scr[...] += jax.lax.dot_general(
        s, y, (((0,), (0,)), ((), ())),
        preferred_element_type=jnp.float32)
    ones_dv = jnp.ones((RB, 128), jnp.float32)
    dv_scr[...] += jax.lax.dot_general(
        s, ones_dv, (((0,), (0,)), ((), ())),
        preferred_element_type=jnp.float32)

    @pl.when(ib == NB - 1)
    def _finish():
        dv = dv_scr[:, 0:1]
        z = z_scr[...]
        o = jnp.where(dv > 0.0, z / (jnp.float32(K) * dv), 0.0)
        o = o + bias_ref[0]
        h = jnp.where(o >= 0.0, o, jnp.float32(0.01) * o)
        feat = jnp.mean(h, axis=0, keepdims=True)
        contrib = jnp.dot(feat, wfc_ref[0],
                          preferred_element_type=jnp.float32)

        @pl.when(l == 0)
        def _first():
            out_ref[0] = bfc_ref[0] + contrib

        @pl.when(l > 0)
        def _rest():
            out_ref[0] += contrib


@jax.jit
def _run(xs, thetas, biases, wfc, bfc):
    grid = (B, NUM_LEVEL, NB)
    return pl.pallas_call(
        _body,
        grid=grid,
        in_specs=[
            pl.BlockSpec((1, 1, RB, D), lambda b, l, ib: (l, b, ib, 0)),
            pl.BlockSpec((1, 1, N, D), lambda b, l, ib: (l, b, 0, 0)),
            pl.BlockSpec((1, D, HID), lambda b, l, ib: (l, 0, 0)),
            pl.BlockSpec((1, 1, HID), lambda b, l, ib: (l, 0, 0)),
            pl.BlockSpec((1, HID, TDIM), lambda b, l, ib: (l, 0, 0)),
            pl.BlockSpec((1, 1, TDIM), lambda b, l, ib: (0, 0, 0)),
        ],
        out_specs=pl.BlockSpec((1, 1, TDIM), lambda b, l, ib: (b, 0, 0)),
        out_shape=jax.ShapeDtypeStruct((B, 1, TDIM), jnp.float32),
        scratch_shapes=[
            pltpu.VMEM((1, N), jnp.float32),
            pltpu.VMEM((N, HID), jnp.float32),
            pltpu.VMEM((N, HID), jnp.float32),
            pltpu.VMEM((N, 128), jnp.float32),
        ],
    )(xs, xs, thetas, biases, wfc, bfc)


def kernel(x0, x1, x2, c0, c1, c2, theta0, bias0, theta1, bias1,
           theta2, bias2, W_fc, b_fc):
    xs = jnp.stack([x0, x1, x2])
    thetas = jnp.stack([theta0, theta1, theta2])
    biases = jnp.stack([bias0, bias1, bias2]).reshape(NUM_LEVEL, 1, HID)
    wfc = W_fc.reshape(NUM_LEVEL, HID, TDIM)
    bfc = b_fc.reshape(1, 1, TDIM)
    out = _run(xs, thetas, biases, wfc, bfc)
    return out.reshape(B, TDIM)
